# baseline (speedup 1.0000x reference)
import functools
import jax
import jax.numpy as jnp
from jax import lax
from jax.experimental import pallas as pl
from jax.experimental.pallas import tpu as pltpu
from jax.experimental.pallas import tpu_sc as plsc

_NB = 7
_RA = 4
_CH = 8192


def _sc_body(nslab, wper, x_hbm, m_hbm, z_hbm, out_hbm,
             mv, bufs, rsems, wsems):
    wid = lax.axis_index("s") * 2 + lax.axis_index("c")
    slab = wid // wper
    qsz = x_hbm.shape[0] // (nslab * wper)
    base = wid * qsz
    nch = qsz // _CH

    def src(i):
        return x_hbm.at[pl.ds(base + i * _CH, _CH)]

    def dst(i):
        return out_hbm.at[pl.ds(base + i * _CH, _CH)]

    pltpu.sync_copy(m_hbm, mv)
    sel = mv[pl.ds(slab, 1)][0]

    @pl.when(sel != 0)
    def _copy():
        def rd(i):
            j = i % _NB
            return pltpu.make_async_copy(src(i), bufs[j], rsems[j])

        def wr(i):
            j = i % _NB
            return pltpu.make_async_copy(bufs[j], dst(i), wsems[j])

        for i in range(_RA):
            rd(i).start()
        for i in range(nch):
            rd(i).wait()
            wr(i).start()
            n = i + _RA
            if n < nch:
                if n >= _NB:
                    wr(n - _NB).wait()
                rd(n).start()
        for i in range(nch - _NB, nch):
            wr(i).wait()

    @pl.when(sel == 0)
    def _zero():
        pltpu.sync_copy(z_hbm, bufs[0])
        for i in range(nch):
            pltpu.make_async_copy(bufs[0], dst(i), wsems[0]).start()
        for i in range(nch):
            pltpu.make_async_copy(bufs[0], dst(i), wsems[0]).wait()


def kernel(x, mask):
    B, K, C, H, W, Z = x.shape
    wper = 32 // (B * K)
    m16 = jnp.pad(mask.reshape(B * K).astype(jnp.int32), (0, 16 - B * K))
    zrow = jnp.zeros((_CH,), jnp.float32)
    xf = x.reshape(B * K * C * H * W * Z)

    mesh = plsc.VectorSubcoreMesh(core_axis_name="c", subcore_axis_name="s")
    fn = functools.partial(
        pl.kernel,
        mesh=mesh,
        out_type=jax.ShapeDtypeStruct((B * K * C * H * W * Z,), x.dtype),
        scratch_types=[
            pltpu.VMEM((16,), jnp.int32),
            [pltpu.VMEM((_CH,), jnp.float32) for _ in range(_NB)],
            [pltpu.SemaphoreType.DMA for _ in range(_NB)],
            [pltpu.SemaphoreType.DMA for _ in range(_NB)],
        ],
    )(functools.partial(_sc_body, B * K, wper))
    return fn(xf, m16, zrow).reshape(B, K * C, H, W, Z)


# speedup vs baseline: 3.0662x; 3.0662x over previous
import functools
import jax
import jax.numpy as jnp
from jax import lax
from jax.experimental import pallas as pl
from jax.experimental.pallas import tpu as pltpu
from jax.experimental.pallas import tpu_sc as plsc

_NB = 3
_RA = 2


def _sc_body(B, K, C, H, W, Z,
             x_hbm, m_hbm, z_hbm, out_hbm,
             mv, bufs, rsems, wsems):
    wid = lax.axis_index("s") * 2 + lax.axis_index("c")
    slab = wid // 4
    q = wid % 4
    b = slab // K
    kk = slab % K
    cq = C // 4
    c0 = q * cq
    hh = H // 16
    hper = H // hh
    nch = cq * hper

    def src(i):
        c, h = divmod(i, hper)
        return x_hbm.at[b, kk, c0 + c, pl.ds(h * hh, hh)]

    def dst(i):
        c, h = divmod(i, hper)
        return out_hbm.at[b, (kk * C + c0 + c), pl.ds(h * hh, hh)]

    pltpu.sync_copy(m_hbm, mv)
    sel = mv[pl.ds(slab, 1)][0]

    @pl.when(sel != 0)
    def _copy():
        def rd(i):
            j = i % _NB
            return pltpu.make_async_copy(src(i), bufs[j], rsems[j])

        def wr(i):
            j = i % _NB
            return pltpu.make_async_copy(bufs[j], dst(i), wsems[j])

        for i in range(_RA):
            rd(i).start()
        for i in range(nch):
            rd(i).wait()
            wr(i).start()
            n = i + _RA
            if n < nch:
                if n >= _NB:
                    wr(n - _NB).wait()
                rd(n).start()
        for i in range(nch - _NB, nch):
            wr(i).wait()

    @pl.when(sel == 0)
    def _zero():
        pltpu.sync_copy(z_hbm, bufs[0])
        for i in range(nch):
            pltpu.make_async_copy(bufs[0], dst(i), wsems[0]).start()
        for i in range(nch):
            pltpu.make_async_copy(bufs[0], dst(i), wsems[0]).wait()


def kernel(x, mask):
    B, K, C, H, W, Z = x.shape
    hh = H // 16
    m16 = jnp.pad(mask.reshape(B * K).astype(jnp.int32), (0, 16 - B * K))
    zrow = jnp.zeros((hh, W, Z), jnp.float32)

    mesh = plsc.VectorSubcoreMesh(core_axis_name="c", subcore_axis_name="s")
    fn = functools.partial(
        pl.kernel,
        mesh=mesh,
        out_type=jax.ShapeDtypeStruct((B, K * C, H, W, Z), x.dtype),
        scratch_types=[
            pltpu.VMEM((16,), jnp.int32),
            [pltpu.VMEM((hh, W, Z), jnp.float32) for _ in range(_NB)],
            [pltpu.SemaphoreType.DMA for _ in range(_NB)],
            [pltpu.SemaphoreType.DMA for _ in range(_NB)],
        ],
    )(functools.partial(_sc_body, B, K, C, H, W, Z))
    return fn(x, m16, zrow)
